# baseline (speedup 1.0000x reference)
import functools

import jax
import jax.numpy as jnp
from jax import lax
from jax.experimental import pallas as pl
from jax.experimental.pallas import tpu as pltpu
from jax.experimental.pallas import tpu_sc as plsc

NUM_EMB = 100000
GENEPT_DIM = 1536
EMB_DIM = 128
EPS = 1e-5

ROW_BLOCK = 800
CHUNK = 128
NSLOT = 4
TC_CHUNK = 1024
N_TC = 196608


def _project_ln_block(t_ref, w_ref, b_ref, g_ref, be_ref, o_ref):
    y = jnp.dot(t_ref[...], w_ref[...],
                preferred_element_type=jnp.float32,
                precision=lax.Precision.DEFAULT)
    y = y + b_ref[...]
    mu = jnp.mean(y, axis=1, keepdims=True)
    d = y - mu
    var = jnp.mean(d * d, axis=1, keepdims=True)
    o_ref[...] = d * lax.rsqrt(var + EPS) * g_ref[...] + be_ref[...]


def _make_projected_table(table, W, b, gamma, beta):
    grid = NUM_EMB // ROW_BLOCK
    return pl.pallas_call(
        _project_ln_block,
        grid=(grid,),
        in_specs=[
            pl.BlockSpec((ROW_BLOCK, GENEPT_DIM), lambda i: (i, 0)),
            pl.BlockSpec((GENEPT_DIM, EMB_DIM), lambda i: (0, 0)),
            pl.BlockSpec((1, EMB_DIM), lambda i: (0, 0)),
            pl.BlockSpec((1, EMB_DIM), lambda i: (0, 0)),
            pl.BlockSpec((1, EMB_DIM), lambda i: (0, 0)),
        ],
        out_specs=pl.BlockSpec((ROW_BLOCK, EMB_DIM), lambda i: (i, 0)),
        out_shape=jax.ShapeDtypeStruct((NUM_EMB, EMB_DIM), jnp.float32),
    )(table, W, b.reshape(1, EMB_DIM), gamma.reshape(1, EMB_DIM),
      beta.reshape(1, EMB_DIM))


def _tc_gather_kernel(idx_ref, p_any, o_ref, sem):
    base = pl.program_id(0) * TC_CHUNK

    def issue(i, c):
        r = idx_ref[base + i]
        pltpu.make_async_copy(p_any.at[pl.ds(r, 1)],
                              o_ref.at[pl.ds(i, 1)], sem).start()
        return c

    lax.fori_loop(0, TC_CHUNK, issue, 0)

    def drain(i, c):
        pltpu.make_async_copy(p_any.at[pl.ds(0, 1)],
                              o_ref.at[pl.ds(0, 1)], sem).wait()
        return c

    lax.fori_loop(0, TC_CHUNK, drain, 0)


def _tc_gather(P, idx_tc, n_tc):
    return pl.pallas_call(
        _tc_gather_kernel,
        grid_spec=pltpu.PrefetchScalarGridSpec(
            num_scalar_prefetch=1,
            grid=(n_tc // TC_CHUNK,),
            in_specs=[pl.BlockSpec(memory_space=pl.ANY)],
            out_specs=pl.BlockSpec((TC_CHUNK, EMB_DIM), lambda i, idx: (i, 0)),
            scratch_shapes=[pltpu.SemaphoreType.DMA],
        ),
        out_shape=jax.ShapeDtypeStruct((n_tc, EMB_DIM), jnp.float32),
    )(idx_tc, P)


def _sc_gather(P, x2d, n_tokens, n_out):
    info = plsc.get_sparse_core_info()
    nw = info.num_cores * info.num_subcores
    n_chunks = n_tokens // CHUNK
    cpw = n_chunks // nw
    mesh = plsc.VectorSubcoreMesh(core_axis_name="c", subcore_axis_name="s")

    @functools.partial(
        pl.kernel, mesh=mesh,
        out_type=jax.ShapeDtypeStruct((n_out, EMB_DIM), jnp.float32),
        scratch_types=[
            pltpu.VMEM((cpw, CHUNK), jnp.int32),
            *([pltpu.VMEM((CHUNK, EMB_DIM), jnp.float32)] * NSLOT),
            *([pltpu.SemaphoreType.DMA] * (2 * NSLOT)),
        ],
    )
    def k(p_hbm, x_hbm, out_hbm, idx_v, *rest):
        bufs = rest[:NSLOT]
        gsems = rest[NSLOT:2 * NSLOT]
        ssems = rest[2 * NSLOT:]
        wid = lax.axis_index("s") * info.num_cores + lax.axis_index("c")
        c0 = wid * cpw
        pltpu.sync_copy(x_hbm.at[pl.ds(c0, cpw)], idx_v)

        def start(j, slot):
            pltpu.async_copy(p_hbm.at[idx_v.at[j]], bufs[slot], gsems[slot])

        def wait_gather(slot):
            pltpu.make_async_copy(p_hbm.at[pl.ds(0, CHUNK)], bufs[slot],
                                  gsems[slot]).wait()

        def store(j, slot):
            pltpu.async_copy(bufs[slot],
                             out_hbm.at[pl.ds((c0 + j) * CHUNK, CHUNK)],
                             ssems[slot])

        def wait_store(slot):
            pltpu.make_async_copy(bufs[slot],
                                  out_hbm.at[pl.ds(0, CHUNK)],
                                  ssems[slot]).wait()

        for s in range(NSLOT):
            start(s, s)

        def body(g, carry):
            j = g * NSLOT
            for t in range(NSLOT):
                wait_gather(t)
                store(j + t, t)
            for t in range(NSLOT):
                wait_store(t)
                start(j + t + NSLOT, t)
            return carry

        lax.fori_loop(0, cpw // NSLOT - 1, body, 0)
        j = cpw - NSLOT
        for t in range(NSLOT):
            wait_gather(t)
            store(j + t, t)
        for t in range(NSLOT):
            wait_store(t)

    return k(P, x2d)


def kernel(x, table, W, b, gamma, beta):
    P = _make_projected_table(table.astype(jnp.float32),
                              W.astype(jnp.float32),
                              b.astype(jnp.float32),
                              gamma.astype(jnp.float32),
                              beta.astype(jnp.float32))
    bsz, seq = x.shape
    n = bsz * seq
    xf = x.astype(jnp.int32).reshape(n)
    n_sc = n - N_TC
    x2d = xf[:n_sc].reshape(n_sc // CHUNK, CHUNK)
    out_sc = _sc_gather(P, x2d, n_sc, n)
    out_tc = _tc_gather(P, xf[n_sc:], N_TC)
    out = lax.dynamic_update_slice(out_sc, out_tc, (n_sc, 0))
    return out.reshape(bsz, seq, EMB_DIM)


# speedup vs baseline: 5.6399x; 5.6399x over previous
import functools

import jax
import jax.numpy as jnp
from jax import lax
from jax.experimental import pallas as pl
from jax.experimental.pallas import tpu as pltpu
from jax.experimental.pallas import tpu_sc as plsc

NUM_EMB = 100000
GENEPT_DIM = 1536
EMB_DIM = 128
EPS = 1e-5

ROW_BLOCK = 2000
CHUNK = 128


def _project_ln_block(t_ref, w_ref, b_ref, g_ref, be_ref, o_ref):
    y = jnp.dot(t_ref[...], w_ref[...],
                preferred_element_type=jnp.float32,
                precision=lax.Precision.DEFAULT)
    y = y + b_ref[...]
    mu = jnp.mean(y, axis=1, keepdims=True)
    d = y - mu
    var = jnp.mean(d * d, axis=1, keepdims=True)
    o_ref[...] = d * lax.rsqrt(var + EPS) * g_ref[...] + be_ref[...]


def _make_projected_table(table, W, b, gamma, beta):
    grid = NUM_EMB // ROW_BLOCK
    return pl.pallas_call(
        _project_ln_block,
        grid=(grid,),
        in_specs=[
            pl.BlockSpec((ROW_BLOCK, GENEPT_DIM), lambda i: (i, 0)),
            pl.BlockSpec((GENEPT_DIM, EMB_DIM), lambda i: (0, 0)),
            pl.BlockSpec((1, EMB_DIM), lambda i: (0, 0)),
            pl.BlockSpec((1, EMB_DIM), lambda i: (0, 0)),
            pl.BlockSpec((1, EMB_DIM), lambda i: (0, 0)),
        ],
        out_specs=pl.BlockSpec((ROW_BLOCK, EMB_DIM), lambda i: (i, 0)),
        out_shape=jax.ShapeDtypeStruct((NUM_EMB, EMB_DIM), jnp.float32),
    )(table, W, b.reshape(1, EMB_DIM), gamma.reshape(1, EMB_DIM),
      beta.reshape(1, EMB_DIM))


def _sc_gather(P, x2d, n_tokens):
    info = plsc.get_sparse_core_info()
    nw = info.num_cores * info.num_subcores
    n_chunks = n_tokens // CHUNK
    cpw = n_chunks // nw
    mesh = plsc.VectorSubcoreMesh(core_axis_name="c", subcore_axis_name="s")

    @functools.partial(
        pl.kernel, mesh=mesh,
        out_type=jax.ShapeDtypeStruct((n_tokens, EMB_DIM), jnp.float32),
        scratch_types=[
            pltpu.VMEM((cpw, CHUNK), jnp.int32),
            pltpu.VMEM((CHUNK, EMB_DIM), jnp.float32),
            pltpu.VMEM((CHUNK, EMB_DIM), jnp.float32),
            pltpu.SemaphoreType.DMA,
            pltpu.SemaphoreType.DMA,
        ],
    )
    def k(p_hbm, x_hbm, out_hbm, idx_v, buf0, buf1, sem0, sem1):
        wid = lax.axis_index("s") * info.num_cores + lax.axis_index("c")
        c0 = wid * cpw
        pltpu.sync_copy(x_hbm.at[pl.ds(c0, cpw)], idx_v)
        bufs = (buf0, buf1)
        sems = (sem0, sem1)

        def start(j, slot):
            pltpu.async_copy(p_hbm.at[idx_v.at[j]], bufs[slot], sems[slot])

        def finish(j, slot):
            pltpu.make_async_copy(p_hbm.at[pl.ds(0, CHUNK)], bufs[slot],
                                  sems[slot]).wait()
            pltpu.sync_copy(bufs[slot],
                            out_hbm.at[pl.ds((c0 + j) * CHUNK, CHUNK)])

        start(0, 0)
        start(1, 1)

        def body(g, carry):
            j = g * 2
            finish(j, 0)
            start(j + 2, 0)
            finish(j + 1, 1)
            start(j + 3, 1)
            return carry

        lax.fori_loop(0, cpw // 2 - 1, body, 0)
        finish(cpw - 2, 0)
        finish(cpw - 1, 1)

    return k(P, x2d)


def kernel(x, table, W, b, gamma, beta):
    P = _make_projected_table(table.astype(jnp.float32),
                              W.astype(jnp.float32),
                              b.astype(jnp.float32),
                              gamma.astype(jnp.float32),
                              beta.astype(jnp.float32))
    bsz, seq = x.shape
    n = bsz * seq
    x2d = x.astype(jnp.int32).reshape(n // CHUNK, CHUNK)
    out = _sc_gather(P, x2d, n)
    return out.reshape(bsz, seq, EMB_DIM)
